# baseline (speedup 1.0000x reference)
import jax
import jax.numpy as jnp
from jax import lax
from jax.experimental import pallas as pl
from jax.experimental.pallas import tpu as pltpu
from jax.experimental.pallas import tpu_sc as plsc

_NC = 2
_NS = 16
_NW = _NC * _NS
_RPG = 8


_NCHUNK = 4


def kernel(X, table):
    B, H = X.shape
    V, D = table.shape
    Bc = B // _NCHUNK
    rows_per_w = Bc // _NW
    ngroups = rows_per_w // _RPG
    assert Bc % (_NW * _RPG) == 0 and ngroups % 2 == 0

    Xi = X.astype(jnp.int32)

    mesh = plsc.VectorSubcoreMesh(core_axis_name="c", subcore_axis_name="s")

    @pl.kernel(
        out_type=jax.ShapeDtypeStruct((Bc, H, D), table.dtype),
        mesh=mesh,
        scratch_types=[
            pltpu.VMEM((rows_per_w, H), jnp.int32),
            pltpu.VMEM((_RPG, H, D), table.dtype),
            pltpu.VMEM((_RPG, H, D), table.dtype),
            pltpu.SemaphoreType.DMA,
            pltpu.SemaphoreType.DMA,
            pltpu.SemaphoreType.DMA,
            pltpu.SemaphoreType.DMA,
        ],
    )
    def gather_kernel(tab_hbm, idx_hbm, out_hbm,
                      idx_v, buf_a, buf_b, g_a, g_b, o_a, o_b):
        wid = lax.axis_index("c") * _NS + lax.axis_index("s")
        rowbase = wid * rows_per_w

        pltpu.sync_copy(idx_hbm.at[pl.ds(rowbase, rows_per_w)], idx_v)

        def fire_gather(g, buf, sem):
            for i in range(_RPG):
                pltpu.async_copy(
                    tab_hbm.at[idx_v.at[g * _RPG + i]], buf.at[i], sem)

        def wait_gather(buf, sem):
            pltpu.make_async_copy(out_hbm.at[pl.ds(0, _RPG)], buf, sem).wait()

        def fire_out(g, buf, sem):
            pltpu.async_copy(
                buf, out_hbm.at[pl.ds(rowbase + g * _RPG, _RPG)], sem)

        def wait_out(g, buf, sem):
            pltpu.make_async_copy(
                buf, out_hbm.at[pl.ds(rowbase + g * _RPG, _RPG)], sem).wait()

        fire_gather(0, buf_a, g_a)
        fire_gather(1, buf_b, g_b)

        @pl.loop(0, ngroups, step=2)
        def _(g0):
            wait_gather(buf_a, g_a)
            fire_out(g0, buf_a, o_a)
            wait_out(g0, buf_a, o_a)

            @pl.when(g0 + 2 < ngroups)
            def _():
                fire_gather(g0 + 2, buf_a, g_a)

            wait_gather(buf_b, g_b)
            fire_out(g0 + 1, buf_b, o_b)

            @pl.when(g0 + 3 < ngroups)
            def _():
                wait_out(g0 + 1, buf_b, o_b)
                fire_gather(g0 + 3, buf_b, g_b)

        wait_out(ngroups - 1, buf_b, o_b)

    chunks = [
        gather_kernel(table, lax.slice(Xi, (c * Bc, 0), ((c + 1) * Bc, H)))
        for c in range(_NCHUNK)
    ]
    return jnp.concatenate(chunks, axis=0)


# speedup vs baseline: 2.9381x; 2.9381x over previous
import jax
import jax.numpy as jnp
from jax import lax
from jax.experimental import pallas as pl
from jax.experimental.pallas import tpu as pltpu
from jax.experimental.pallas import tpu_sc as plsc

_NC = 2
_NS = 16
_NW = _NC * _NS
_RPG = 8


def kernel(X, table):
    B, H = X.shape
    V, D = table.shape
    rows_per_w = B // _NW
    ngroups = rows_per_w // _RPG
    assert B % (_NW * _RPG) == 0 and ngroups % 2 == 0

    Xi = X.astype(jnp.int32)

    mesh = plsc.VectorSubcoreMesh(core_axis_name="c", subcore_axis_name="s")

    @pl.kernel(
        out_type=jax.ShapeDtypeStruct((H, B, D), table.dtype),
        mesh=mesh,
        scratch_types=[
            pltpu.VMEM((rows_per_w, H), jnp.int32),
            pltpu.VMEM((H, _RPG, D), table.dtype),
            pltpu.VMEM((H, _RPG, D), table.dtype),
            pltpu.SemaphoreType.DMA,
            pltpu.SemaphoreType.DMA,
            pltpu.SemaphoreType.DMA,
            pltpu.SemaphoreType.DMA,
        ],
    )
    def gather_kernel(tab_hbm, idx_hbm, out_hbm,
                      idx_v, buf_a, buf_b, g_a, g_b, o_a, o_b):
        wid = lax.axis_index("c") * _NS + lax.axis_index("s")
        rowbase = wid * rows_per_w

        pltpu.sync_copy(idx_hbm.at[pl.ds(rowbase, rows_per_w)], idx_v)

        def fire_gather(g, buf, sem):
            for i in range(_RPG):
                pltpu.async_copy(
                    tab_hbm.at[idx_v.at[g * _RPG + i]], buf.at[:, i, :], sem)

        def wait_gather(buf, sem):
            pltpu.make_async_copy(
                out_hbm.at[:, pl.ds(0, _RPG), :], buf, sem).wait()

        def fire_out(g, buf, sem):
            pltpu.async_copy(
                buf, out_hbm.at[:, pl.ds(rowbase + g * _RPG, _RPG), :], sem)

        def wait_out(g, buf, sem):
            pltpu.make_async_copy(
                buf, out_hbm.at[:, pl.ds(rowbase + g * _RPG, _RPG), :],
                sem).wait()

        fire_gather(0, buf_a, g_a)
        fire_gather(1, buf_b, g_b)

        @pl.loop(0, ngroups, step=2)
        def _(g0):
            wait_gather(buf_a, g_a)
            fire_out(g0, buf_a, o_a)
            wait_out(g0, buf_a, o_a)

            @pl.when(g0 + 2 < ngroups)
            def _():
                fire_gather(g0 + 2, buf_a, g_a)

            wait_gather(buf_b, g_b)
            fire_out(g0 + 1, buf_b, o_b)

            @pl.when(g0 + 3 < ngroups)
            def _():
                wait_out(g0 + 1, buf_b, o_b)
                fire_gather(g0 + 3, buf_b, g_b)

        wait_out(ngroups - 1, buf_b, o_b)

    out_t = gather_kernel(table, Xi)
    return jnp.transpose(out_t, (1, 0, 2))


# speedup vs baseline: 3.1682x; 1.0783x over previous
import jax
import jax.numpy as jnp
from jax import lax
from jax.experimental import pallas as pl
from jax.experimental.pallas import tpu as pltpu
from jax.experimental.pallas import tpu_sc as plsc

_NC = 2
_NS = 16
_NW = _NC * _NS


def kernel(X, table):
    B, H = X.shape
    V, D = table.shape
    cols_per_w = B // _NW
    assert B % _NW == 0 and H % 2 == 0

    Xt = X.astype(jnp.int32).T

    mesh = plsc.VectorSubcoreMesh(core_axis_name="c", subcore_axis_name="s")

    @pl.kernel(
        out_type=jax.ShapeDtypeStruct((H, B, D), table.dtype),
        mesh=mesh,
        scratch_types=[
            pltpu.VMEM((H, cols_per_w), jnp.int32),
            pltpu.VMEM((cols_per_w, D), table.dtype),
            pltpu.VMEM((cols_per_w, D), table.dtype),
            pltpu.SemaphoreType.DMA,
            pltpu.SemaphoreType.DMA,
            pltpu.SemaphoreType.DMA,
            pltpu.SemaphoreType.DMA,
        ],
    )
    def gather_kernel(tab_hbm, idx_hbm, out_hbm,
                      idx_v, buf_a, buf_b, g_a, g_b, o_a, o_b):
        wid = lax.axis_index("c") * _NS + lax.axis_index("s")
        colbase = wid * cols_per_w

        pltpu.sync_copy(
            idx_hbm.at[:, pl.ds(colbase, cols_per_w)], idx_v)

        def fire_gather(h, buf, sem):
            pltpu.async_copy(tab_hbm.at[idx_v.at[h]], buf, sem)

        def wait_gather(buf, sem):
            pltpu.make_async_copy(
                tab_hbm.at[pl.ds(0, cols_per_w)], buf, sem).wait()

        def fire_out(h, buf, sem):
            pltpu.async_copy(
                buf, out_hbm.at[h, pl.ds(colbase, cols_per_w)], sem)

        def wait_out(h, buf, sem):
            pltpu.make_async_copy(
                buf, out_hbm.at[h, pl.ds(colbase, cols_per_w)], sem).wait()

        fire_gather(0, buf_a, g_a)
        fire_gather(1, buf_b, g_b)

        @pl.loop(0, H, step=2)
        def _(h0):
            wait_gather(buf_a, g_a)
            fire_out(h0, buf_a, o_a)
            wait_out(h0, buf_a, o_a)

            @pl.when(h0 + 2 < H)
            def _():
                fire_gather(h0 + 2, buf_a, g_a)

            wait_gather(buf_b, g_b)
            fire_out(h0 + 1, buf_b, o_b)

            @pl.when(h0 + 3 < H)
            def _():
                wait_out(h0 + 1, buf_b, o_b)
                fire_gather(h0 + 3, buf_b, g_b)

        wait_out(H - 1, buf_b, o_b)

    out_t = gather_kernel(table, Xt)
    return jnp.transpose(out_t, (1, 0, 2))


# speedup vs baseline: 3.2149x; 1.0147x over previous
import jax
import jax.numpy as jnp
from jax import lax
from jax.experimental import pallas as pl
from jax.experimental.pallas import tpu as pltpu
from jax.experimental.pallas import tpu_sc as plsc

_NC = 2
_NS = 16
_NW = _NC * _NS


def kernel(X, table):
    B, H = X.shape
    V, D = table.shape
    cols_per_w = B // _NW
    assert B % _NW == 0 and H % 2 == 0

    Xt = X.astype(jnp.int32).T

    mesh = plsc.VectorSubcoreMesh(core_axis_name="c", subcore_axis_name="s")

    @pl.kernel(
        out_type=jax.ShapeDtypeStruct((H, B, D), table.dtype),
        mesh=mesh,
        scratch_types=[
            pltpu.VMEM((H, cols_per_w), jnp.int32),
            pltpu.VMEM((cols_per_w, D), table.dtype),
            pltpu.VMEM((cols_per_w, D), table.dtype),
            pltpu.VMEM((cols_per_w, D), table.dtype),
            pltpu.VMEM((cols_per_w, D), table.dtype),
            pltpu.SemaphoreType.DMA,
            pltpu.SemaphoreType.DMA,
            pltpu.SemaphoreType.DMA,
            pltpu.SemaphoreType.DMA,
            pltpu.SemaphoreType.DMA,
            pltpu.SemaphoreType.DMA,
            pltpu.SemaphoreType.DMA,
            pltpu.SemaphoreType.DMA,
        ],
    )
    def gather_kernel(tab_hbm, idx_hbm, out_hbm,
                      idx_v, b0, b1, b2, b3,
                      g0, g1, g2, g3, o0, o1, o2, o3):
        bufs = (b0, b1, b2, b3)
        gsems = (g0, g1, g2, g3)
        osems = (o0, o1, o2, o3)
        wid = lax.axis_index("c") * _NS + lax.axis_index("s")
        colbase = wid * cols_per_w

        pltpu.sync_copy(
            idx_hbm.at[:, pl.ds(colbase, cols_per_w)], idx_v)

        def fire_gather(h, buf, sem):
            pltpu.async_copy(tab_hbm.at[idx_v.at[h]], buf, sem)

        def wait_gather(buf, sem):
            pltpu.make_async_copy(
                tab_hbm.at[pl.ds(0, cols_per_w)], buf, sem).wait()

        def fire_out(h, buf, sem):
            pltpu.async_copy(
                buf, out_hbm.at[h, pl.ds(colbase, cols_per_w)], sem)

        def wait_out(h, buf, sem):
            pltpu.make_async_copy(
                buf, out_hbm.at[h, pl.ds(colbase, cols_per_w)], sem).wait()

        for i in range(4):
            fire_gather(i, bufs[i], gsems[i])

        @pl.loop(0, H - 2, step=4)
        def _(h0):
            for i in range(4):
                h = h0 + i
                wait_gather(bufs[i], gsems[i])
                fire_out(h, bufs[i], osems[i])
                prev = (i - 1) % 4

                @pl.when((h >= 1) & (h + 3 < H))
                def _():
                    wait_out(h - 1, bufs[prev], osems[prev])
                    fire_gather(h + 3, bufs[prev], gsems[prev])

        wait_gather(bufs[0], gsems[0])
        fire_out(H - 2, bufs[0], osems[0])
        wait_out(H - 4, bufs[2], osems[2])
        wait_out(H - 3, bufs[3], osems[3])
        wait_gather(bufs[1], gsems[1])
        fire_out(H - 1, bufs[1], osems[1])
        wait_out(H - 2, bufs[0], osems[0])
        wait_out(H - 1, bufs[1], osems[1])

    out_t = gather_kernel(table, Xt)
    return jnp.transpose(out_t, (1, 0, 2))
